# baseline (speedup 1.0000x reference)
import functools

import jax
import jax.numpy as jnp
from jax import lax
from jax.experimental import pallas as pl
from jax.experimental.pallas import tpu as pltpu
from jax.experimental.pallas import tpu_sc as plsc

N = 100000
E = 6400000
G = 512
H = 32
NP = 102400
NB = 50
RB = 2048
NCHUNK = 2000
NBLK = 50000
CE = 3200
SUB = 25
CEC = 640
SUBC = 5
NCHC = 625
NPT = NP // 16
F32 = jnp.float32
I32 = jnp.int32

_mesh = plsc.VectorSubcoreMesh(core_axis_name="c", subcore_axis_name="s",
                               num_cores=2, num_subcores=16)


@functools.partial(
    pl.kernel, mesh=_mesh,
    compiler_params=pltpu.CompilerParams(use_tc_tiling_on_sc=False, needs_layout_passes=False),
    out_type=(jax.ShapeDtypeStruct((2, NP), F32),
              jax.ShapeDtypeStruct((2, NP), F32)),
    scratch_types=[
        pltpu.VMEM_SHARED((NP,), F32),
        pltpu.VMEM_SHARED((NP,), F32),
        pltpu.VMEM_SHARED((NP,), F32),
        pltpu.VMEM((2, SUB, 128), I32),
        pltpu.VMEM((2, SUB, 128), I32),
        pltpu.VMEM((2, SUB, 128), F32),
        pltpu.VMEM((128,), F32),
        pltpu.SemaphoreType.DMA,
        pltpu.SemaphoreType.DMA,
        pltpu.SemaphoreType.DMA,
        pltpu.SemaphoreType.DMA,
    ])
def _sc_stats(edge2, xpad, zerosT, ones128, s1_out, deg_out,
              x_sh, s1_sh, deg_sh, src_v, dst_v, xs_v, ones_v,
              lsem, gsem, asem, bsem):
    c = lax.axis_index("c")
    s = lax.axis_index("s")

    @pl.when(s == 0)
    def _():
        pltpu.sync_copy(xpad, x_sh)

    pltpu.sync_copy(zerosT, s1_sh.at[pl.ds(s * NPT, NPT)])
    pltpu.sync_copy(zerosT, deg_sh.at[pl.ds(s * NPT, NPT)])
    pltpu.sync_copy(ones128, ones_v)
    plsc.subcore_barrier()

    w = s * 2 + c
    start = w * 62 + jnp.minimum(w, 16)
    count = 62 + jnp.where(w < 16, 1, 0)

    def chunk_body(t, carry):
        cb = (start + t) * SUB
        pltpu.sync_copy(edge2.at[0, pl.ds(cb, SUB), :], src_v.at[0])
        pltpu.sync_copy(edge2.at[1, pl.ds(cb, SUB), :], dst_v.at[0])
        gds = [pltpu.async_copy(x_sh.at[src_v.at[0, j]], xs_v.at[0, j], gsem)
               for j in range(SUB)]
        for d in gds:
            d.wait()
        sds = []
        for j in range(SUB):
            sds.append(pltpu.async_copy(xs_v.at[0, j], s1_sh.at[dst_v.at[0, j]],
                                        asem, add=True))
            sds.append(pltpu.async_copy(ones_v, deg_sh.at[dst_v.at[0, j]],
                                        bsem, add=True))
        for d in sds:
            d.wait()
        return carry

    lax.fori_loop(0, count, chunk_body, 0)
    plsc.subcore_barrier()
    pltpu.sync_copy(s1_sh.at[pl.ds(s * NPT, NPT)],
                    s1_out.at[c, pl.ds(s * NPT, NPT)])
    pltpu.sync_copy(deg_sh.at[pl.ds(s * NPT, NPT)],
                    deg_out.at[c, pl.ds(s * NPT, NPT)])


def _tc_prep_body(s1p, degp, xp, w1l, b1l, w1r, g1, be1,
                  a_out, deg_out, pqr_out):
    s1 = s1p[0] + s1p[1]
    deg = degp[0] + degp[1]
    a = s1 / jnp.maximum(deg, 1.0)
    b = xp[...]
    n = jnp.float32(N)
    mua = jnp.sum(a) / n
    mub = jnp.sum(b) / n
    va = jnp.sum(a * a) / n - mua * mua
    vb = jnp.sum(b * b) / n - mub * mub
    cab = jnp.sum(a * b) / n - mua * mub
    u = w1l[...]
    v = w1r[...]
    c0 = b1l[...]
    mu = mua * u + mub * v + c0
    var = u * u * va + v * v * vb + 2.0 * u * v * cab
    inv = g1[...] * lax.rsqrt(var + 1e-5)
    p = u * inv
    q = v * inv
    r = (c0 - mu) * inv + be1[...]
    a_out[...] = a
    deg_out[...] = deg
    pqr_out[...] = jnp.concatenate([p, q, r], axis=0)


def _tc_prep(s1p, degp, xp, w1l, b1l, w1r, g1, be1):
    return pl.pallas_call(
        _tc_prep_body,
        out_shape=(jax.ShapeDtypeStruct((800, 128), F32),
                   jax.ShapeDtypeStruct((800, 128), F32),
                   jax.ShapeDtypeStruct((3, H), F32)),
    )(s1p, degp, xp, w1l, b1l, w1r, g1, be1)


NT = N // 16


@functools.partial(
    pl.kernel, mesh=_mesh,
    compiler_params=pltpu.CompilerParams(use_tc_tiling_on_sc=False, needs_layout_passes=False),
    out_type=jax.ShapeDtypeStruct((NP, H), F32),
    scratch_types=[
        pltpu.VMEM_SHARED((N,), F32),
        pltpu.VMEM_SHARED((N,), F32),
        pltpu.VMEM_SHARED((N, 16), F32),
        pltpu.VMEM((3, CEC), I32),
        pltpu.VMEM((3, CEC), I32),
        pltpu.VMEM((2, CEC), F32),
        pltpu.VMEM((2, CEC), F32),
        pltpu.VMEM((CEC, 16), F32),
        pltpu.VMEM((3, H), F32),
        pltpu.SemaphoreType.DMA,
        pltpu.SemaphoreType.DMA,
        pltpu.SemaphoreType.DMA,
    ])
def _sc_agg(edge_flat, a_hbm, b_hbm, pqr, zacc, agg_out,
            a_sh, b_sh, acc_sh, src_v, dst_v, a_v, b_v, vals_v, pqr_v,
            lsem, gsem, ssem):
    c = lax.axis_index("c")
    s = lax.axis_index("s")

    @pl.when(s == 0)
    def _():
        pltpu.sync_copy(a_hbm, a_sh)
        pltpu.sync_copy(b_hbm, b_sh)

    pltpu.sync_copy(pqr, pqr_v)
    pltpu.sync_copy(zacc, acc_sh.at[pl.ds(s * NT, NT), :])
    plsc.subcore_barrier()

    base_f = c * 16
    pv = pqr_v[0, pl.ds(base_f, 16)]
    qv = pqr_v[1, pl.ds(base_f, 16)]
    rv = pqr_v[2, pl.ds(base_f, 16)]
    pk = [pv[k] for k in range(16)]
    qk = [qv[k] for k in range(16)]
    rk = [rv[k] for k in range(16)]
    iota = lax.iota(I32, 16)
    kcols = [jnp.full((16,), k, I32) for k in range(16)]

    def loads(tb, ib):
        e0 = (s * NCHC + tb) * CEC
        return [pltpu.make_async_copy(edge_flat.at[0, pl.ds(e0, CEC)],
                                      src_v.at[ib], lsem),
                pltpu.make_async_copy(edge_flat.at[1, pl.ds(e0, CEC)],
                                      dst_v.at[ib], lsem)]

    def gathers(tb, ib, ab=0):
        del tb
        return [pltpu.make_async_copy(a_sh.at[src_v.at[ib]], a_v.at[ab], gsem),
                pltpu.make_async_copy(b_sh.at[src_v.at[ib]], b_v.at[ab], gsem)]

    def scatters(tb, ib):
        del tb
        return [pltpu.make_async_copy(vals_v, acc_sh.at[dst_v.at[ib]], ssem)]

    def compute(buf):
        def group_body(g, carry2):
            gbase = g * 16
            rows = iota + gbase
            av = a_v[buf, pl.ds(gbase, 16)]
            bv = b_v[buf, pl.ds(gbase, 16)]
            for k in range(16):
                col = jnp.maximum(av * pk[k] + bv * qk[k] + rk[k], 0.0)
                plsc.store_scatter(vals_v, [rows, kcols[k]], col)
            return carry2
        lax.fori_loop(0, CEC // 16, group_body, 0)

    for d in loads(0, 0):
        d.start()
    for d in loads(0, 0):
        d.wait()
    for d in gathers(0, 0):
        d.start()
    for d in loads(1, 1):
        d.start()
    for d in loads(1, 1):
        d.wait()
    for d in gathers(1, 1):
        d.start()
    for d in gathers(0, 0):
        d.wait()
    compute(0)
    for d in scatters(0, 0):
        d.start(add=True)
    for d in loads(2, 2):
        d.start()

    def chunk_body(t, carry):
        ib = lax.rem(t, 3)
        ib1 = lax.rem(t + 1, 3)
        ib2 = lax.rem(t + 2, 3)
        ab = lax.rem(t, 2)
        ab1 = lax.rem(t + 1, 2)
        tn1 = jnp.minimum(t + 1, NCHC - 1)
        tn2 = jnp.minimum(t + 2, NCHC - 1)
        for d in loads(tn1, ib1):
            d.wait()
        for d in gathers(tn1, ib1, ab1):
            d.start()
        for d in scatters(tn1, ib1):
            d.wait()
        for d in gathers(t, ib, ab):
            d.wait()
        compute(ab)
        for d in scatters(t, ib):
            d.start(add=True)
        for d in loads(tn2, ib2):
            d.start()
        return carry

    lax.fori_loop(1, NCHC, chunk_body, 0)
    for d in loads(0, lax.rem(NCHC + 1, 3)):
        d.wait()
    for d in gathers(0, lax.rem(NCHC, 3), lax.rem(NCHC, 2)):
        d.wait()
    for d in scatters(0, lax.rem(NCHC - 1, 3)):
        d.wait()
    plsc.subcore_barrier()
    pltpu.sync_copy(acc_sh.at[pl.ds(s * NT, NT), :],
                    agg_out.at[pl.ds(s * NT, NT), pl.ds(c * 16, 16)])


def _tc_z2_body(agg, deg3, a3, b3, pqr, w2l, w2r, b2l, z2_out, st_out, sacc):
    i = pl.program_id(0)
    deg = jnp.reshape(deg3[...], (RB, 1))
    a = jnp.reshape(a3[...], (RB, 1))
    b = jnp.reshape(b3[...], (RB, 1))
    p = pqr[0:1, :]
    q = pqr[1:2, :]
    r = pqr[2:3, :]
    h1 = jnp.maximum(a * p + b * q + r, 0.0)
    mean2 = agg[...] / jnp.maximum(deg, 1.0)
    z2 = (jnp.dot(mean2, w2l[...], preferred_element_type=F32)
          + jnp.dot(h1, w2r[...], preferred_element_type=F32) + b2l[...])
    valid = (lax.broadcasted_iota(I32, (RB, 1), 0) + i * RB) < N
    z2 = jnp.where(valid, z2, 0.0)
    z2_out[...] = z2

    @pl.when(i == 0)
    def _():
        sacc[...] = jnp.zeros((2, H), F32)

    sacc[0:1, :] += jnp.sum(z2, axis=0, keepdims=True)
    sacc[1:2, :] += jnp.sum(z2 * z2, axis=0, keepdims=True)

    @pl.when(i == NB - 1)
    def _():
        st_out[...] = sacc[...]


def _tc_z2(agg, deg3, a3, b3, pqr, w2l, w2r, b2l):
    return pl.pallas_call(
        _tc_z2_body,
        grid=(NB,),
        in_specs=[
            pl.BlockSpec((RB, H), lambda i: (i, 0)),
            pl.BlockSpec((1, RB, 1), lambda i: (i, 0, 0)),
            pl.BlockSpec((1, RB, 1), lambda i: (i, 0, 0)),
            pl.BlockSpec((1, RB, 1), lambda i: (i, 0, 0)),
            pl.BlockSpec((3, H), lambda i: (0, 0)),
            pl.BlockSpec((H, H), lambda i: (0, 0)),
            pl.BlockSpec((H, H), lambda i: (0, 0)),
            pl.BlockSpec((1, H), lambda i: (0, 0)),
        ],
        out_specs=[
            pl.BlockSpec((RB, H), lambda i: (i, 0)),
            pl.BlockSpec((2, H), lambda i: (0, 0)),
        ],
        out_shape=[jax.ShapeDtypeStruct((NP, H), F32),
                   jax.ShapeDtypeStruct((2, H), F32)],
        scratch_shapes=[pltpu.VMEM((2, H), F32)],
    )(agg, deg3, a3, b3, pqr, w2l, w2r, b2l)


def _tc_pool_body(z2, st, bat3, g2, be2, wg, bg, wo, bo, out,
                  gmax_s, den_s, num_s):
    ph = pl.program_id(0)
    i = pl.program_id(1)
    n = jnp.float32(N)
    mean = st[0:1, :] / n
    var = st[1:2, :] / n - mean * mean
    s2 = g2[...] * lax.rsqrt(var + 1e-5)
    t2 = be2[...] - mean * s2
    h2 = jnp.maximum(z2[...] * s2 + t2, 0.0)
    gate = jnp.dot(h2, wg[...], preferred_element_type=F32) + bg[...]
    bat = jnp.reshape(bat3[...], (RB, 1))
    gid = lax.broadcasted_iota(I32, (RB, G), 1).astype(F32)
    onehot = bat == gid
    onef = onehot.astype(F32)

    @pl.when(ph == 0)
    def _():
        @pl.when(i == 0)
        def _():
            gmax_s[...] = jnp.full((1, G), -jnp.inf, F32)
        masked = jnp.where(onehot, gate, -jnp.inf)
        bm = jnp.max(masked, axis=0, keepdims=True)
        gmax_s[...] = jnp.maximum(gmax_s[...], bm)

    @pl.when(ph == 1)
    def _():
        @pl.when(i == 0)
        def _():
            g0 = gmax_s[...]
            gmax_s[...] = jnp.where(jnp.isfinite(g0), g0, 0.0)
            den_s[...] = jnp.zeros((G, 1), F32)
            num_s[...] = jnp.zeros((G, H), F32)
        rowg = jnp.sum(jnp.where(onehot, gmax_s[...], 0.0),
                       axis=1, keepdims=True)
        e = jnp.exp(gate - rowg)
        e = jnp.where(bat < jnp.float32(G), e, 0.0)
        dn = (((0,), (0,)), ((), ()))
        den_s[...] += lax.dot_general(onef, e, dn, preferred_element_type=F32)
        num_s[...] += lax.dot_general(onef, e * h2, dn,
                                      preferred_element_type=F32)

        @pl.when(i == NB - 1)
        def _():
            pooled = num_s[...] / jnp.maximum(den_s[...], 1e-16)
            logit = jnp.dot(pooled, wo[...], preferred_element_type=F32) + bo[...]
            out[...] = jax.nn.sigmoid(logit)


def _tc_pool(z2, st, bat3, g2, be2, wg, bg, wo, bo):
    return pl.pallas_call(
        _tc_pool_body,
        grid=(2, NB),
        in_specs=[
            pl.BlockSpec((RB, H), lambda p, i: (i, 0)),
            pl.BlockSpec((2, H), lambda p, i: (0, 0)),
            pl.BlockSpec((1, RB, 1), lambda p, i: (i, 0, 0)),
            pl.BlockSpec((1, H), lambda p, i: (0, 0)),
            pl.BlockSpec((1, H), lambda p, i: (0, 0)),
            pl.BlockSpec((H, 1), lambda p, i: (0, 0)),
            pl.BlockSpec((1, 1), lambda p, i: (0, 0)),
            pl.BlockSpec((H, 1), lambda p, i: (0, 0)),
            pl.BlockSpec((1, 1), lambda p, i: (0, 0)),
        ],
        out_specs=pl.BlockSpec((G, 1), lambda p, i: (0, 0)),
        out_shape=jax.ShapeDtypeStruct((G, 1), F32),
        scratch_shapes=[pltpu.VMEM((1, G), F32),
                        pltpu.VMEM((G, 1), F32),
                        pltpu.VMEM((G, H), F32)],
    )(z2, st, bat3, g2, be2, wg, bg, wo, bo)


def kernel(x, edge_index, batch, W1l, b1l, W1r, g1, be1,
           W2l, b2l, W2r, g2, be2, Wg, bg, Wo, bo):
    xf = jnp.reshape(x, (N,))
    xpad = jnp.pad(xf, (0, NP - N))
    edge2 = edge_index.reshape(2, NBLK, 128)
    zerosT = jnp.zeros((NPT,), F32)
    ones128 = jnp.ones((128,), F32)
    zacc = jnp.zeros((NT, 16), F32)

    s1p, degp = _sc_stats(edge2, xpad, zerosT, ones128)

    a_arr, deg_arr, pqr = _tc_prep(
        s1p.reshape(2, 800, 128), degp.reshape(2, 800, 128),
        xpad.reshape(800, 128), W1l, jnp.reshape(b1l, (1, H)), W1r,
        jnp.reshape(g1, (1, H)), jnp.reshape(be1, (1, H)))

    agg = _sc_agg(edge_index, a_arr.reshape(NP)[:N], xf, pqr, zacc)

    deg3 = deg_arr.reshape(NB, RB, 1)
    a3 = a_arr.reshape(NB, RB, 1)
    b3 = xpad.reshape(NB, RB, 1)
    z2, st = _tc_z2(agg, deg3, a3, b3, pqr, W2l, W2r,
                    jnp.reshape(b2l, (1, H)))

    bat3 = jnp.pad(batch, (0, NP - N), constant_values=G).astype(F32)
    bat3 = bat3.reshape(NB, RB, 1)
    out = _tc_pool(z2, st, bat3, jnp.reshape(g2, (1, H)),
                   jnp.reshape(be2, (1, H)), Wg, jnp.reshape(bg, (1, 1)),
                   Wo, jnp.reshape(bo, (1, 1)))
    return out[:, 0]


# speedup vs baseline: 1.4209x; 1.4209x over previous
import functools

import jax
import jax.numpy as jnp
from jax import lax
from jax.experimental import pallas as pl
from jax.experimental.pallas import tpu as pltpu
from jax.experimental.pallas import tpu_sc as plsc

N = 100000
E = 6400000
G = 512
H = 32
NP = 102400
NB = 50
RB = 2048
NCHUNK = 2000
NBLK = 50000
CE = 3200
SUB = 25
CEC = 640
SUBC = 5
NCHC = 625
NPT = NP // 16
F32 = jnp.float32
I32 = jnp.int32

_mesh = plsc.VectorSubcoreMesh(core_axis_name="c", subcore_axis_name="s",
                               num_cores=2, num_subcores=16)


@functools.partial(
    pl.kernel, mesh=_mesh,
    compiler_params=pltpu.CompilerParams(use_tc_tiling_on_sc=False, needs_layout_passes=False),
    out_type=(jax.ShapeDtypeStruct((2, NP), F32),
              jax.ShapeDtypeStruct((2, NP), F32)),
    scratch_types=[
        pltpu.VMEM_SHARED((NP,), F32),
        pltpu.VMEM_SHARED((NP,), F32),
        pltpu.VMEM_SHARED((NP,), F32),
        pltpu.VMEM((2, SUB, 128), I32),
        pltpu.VMEM((2, SUB, 128), I32),
        pltpu.VMEM((2, SUB, 128), F32),
        pltpu.VMEM((128,), F32),
        pltpu.SemaphoreType.DMA,
        pltpu.SemaphoreType.DMA,
        pltpu.SemaphoreType.DMA,
        pltpu.SemaphoreType.DMA,
    ])
def _sc_stats(edge2, xpad, zerosT, ones128, s1_out, deg_out,
              x_sh, s1_sh, deg_sh, src_v, dst_v, xs_v, ones_v,
              lsem, gsem, asem, bsem):
    c = lax.axis_index("c")
    s = lax.axis_index("s")

    @pl.when(s == 0)
    def _():
        pltpu.sync_copy(xpad, x_sh)

    pltpu.sync_copy(zerosT, s1_sh.at[pl.ds(s * NPT, NPT)])
    pltpu.sync_copy(zerosT, deg_sh.at[pl.ds(s * NPT, NPT)])
    pltpu.sync_copy(ones128, ones_v)
    plsc.subcore_barrier()

    w = s * 2 + c
    start = w * 62 + jnp.minimum(w, 16)
    count = 62 + jnp.where(w < 16, 1, 0)

    def chunk_body(t, carry):
        cb = (start + t) * SUB
        pltpu.sync_copy(edge2.at[0, pl.ds(cb, SUB), :], src_v.at[0])
        pltpu.sync_copy(edge2.at[1, pl.ds(cb, SUB), :], dst_v.at[0])
        gds = [pltpu.async_copy(x_sh.at[src_v.at[0, j]], xs_v.at[0, j], gsem)
               for j in range(SUB)]
        for d in gds:
            d.wait()
        sds = []
        for j in range(SUB):
            sds.append(pltpu.async_copy(xs_v.at[0, j], s1_sh.at[dst_v.at[0, j]],
                                        asem, add=True))
            sds.append(pltpu.async_copy(ones_v, deg_sh.at[dst_v.at[0, j]],
                                        bsem, add=True))
        for d in sds:
            d.wait()
        return carry

    lax.fori_loop(0, count, chunk_body, 0)
    plsc.subcore_barrier()
    pltpu.sync_copy(s1_sh.at[pl.ds(s * NPT, NPT)],
                    s1_out.at[c, pl.ds(s * NPT, NPT)])
    pltpu.sync_copy(deg_sh.at[pl.ds(s * NPT, NPT)],
                    deg_out.at[c, pl.ds(s * NPT, NPT)])


def _tc_prep_body(s1p, degp, xp, w1l, b1l, w1r, g1, be1,
                  a_out, deg_out, pqr_out):
    s1 = s1p[0] + s1p[1]
    deg = degp[0] + degp[1]
    a = s1 / jnp.maximum(deg, 1.0)
    b = xp[...]
    n = jnp.float32(N)
    mua = jnp.sum(a) / n
    mub = jnp.sum(b) / n
    va = jnp.sum(a * a) / n - mua * mua
    vb = jnp.sum(b * b) / n - mub * mub
    cab = jnp.sum(a * b) / n - mua * mub
    u = w1l[...]
    v = w1r[...]
    c0 = b1l[...]
    mu = mua * u + mub * v + c0
    var = u * u * va + v * v * vb + 2.0 * u * v * cab
    inv = g1[...] * lax.rsqrt(var + 1e-5)
    p = u * inv
    q = v * inv
    r = (c0 - mu) * inv + be1[...]
    a_out[...] = a
    deg_out[...] = deg
    pqr_out[...] = jnp.concatenate([p, q, r], axis=0)


def _tc_prep(s1p, degp, xp, w1l, b1l, w1r, g1, be1):
    return pl.pallas_call(
        _tc_prep_body,
        out_shape=(jax.ShapeDtypeStruct((800, 128), F32),
                   jax.ShapeDtypeStruct((800, 128), F32),
                   jax.ShapeDtypeStruct((3, H), F32)),
    )(s1p, degp, xp, w1l, b1l, w1r, g1, be1)


NT = N // 16


@functools.partial(
    pl.kernel, mesh=_mesh,
    compiler_params=pltpu.CompilerParams(use_tc_tiling_on_sc=False, needs_layout_passes=False),
    out_type=jax.ShapeDtypeStruct((NP, H), F32),
    scratch_types=[
        pltpu.VMEM_SHARED((N,), F32),
        pltpu.VMEM_SHARED((N,), F32),
        pltpu.VMEM_SHARED((N, 16), F32),
        pltpu.VMEM((3, CEC), I32),
        pltpu.VMEM((3, CEC), I32),
        pltpu.VMEM((2, CEC), F32),
        pltpu.VMEM((2, CEC), F32),
        pltpu.VMEM((CEC, 16), F32),
        pltpu.VMEM((3, H), F32),
        pltpu.SemaphoreType.DMA,
        pltpu.SemaphoreType.DMA,
        pltpu.SemaphoreType.DMA,
    ])
def _sc_agg(edge_flat, a_hbm, b_hbm, pqr, zacc, agg_out,
            a_sh, b_sh, acc_sh, src_v, dst_v, a_v, b_v, vals_v, pqr_v,
            lsem, gsem, ssem):
    c = lax.axis_index("c")
    s = lax.axis_index("s")

    @pl.when(s == 0)
    def _():
        pltpu.sync_copy(a_hbm, a_sh)
        pltpu.sync_copy(b_hbm, b_sh)

    pltpu.sync_copy(pqr, pqr_v)
    pltpu.sync_copy(zacc, acc_sh.at[pl.ds(s * NT, NT), :])
    plsc.subcore_barrier()

    base_f = c * 16
    pv = pqr_v[0, pl.ds(base_f, 16)]
    qv = pqr_v[1, pl.ds(base_f, 16)]
    rv = pqr_v[2, pl.ds(base_f, 16)]
    pk = [pv[k] for k in range(16)]
    qk = [qv[k] for k in range(16)]
    rk = [rv[k] for k in range(16)]
    iota = lax.iota(I32, 16)
    kcols = [jnp.full((16,), k, I32) for k in range(16)]

    def loads(tb, ib):
        e0 = (s * NCHC + tb) * CEC
        return [pltpu.make_async_copy(edge_flat.at[0, pl.ds(e0, CEC)],
                                      src_v.at[ib], lsem),
                pltpu.make_async_copy(edge_flat.at[1, pl.ds(e0, CEC)],
                                      dst_v.at[ib], lsem)]

    def gathers(tb, ib, ab=0):
        del tb
        return [pltpu.make_async_copy(a_sh.at[src_v.at[ib]], a_v.at[ab], gsem),
                pltpu.make_async_copy(b_sh.at[src_v.at[ib]], b_v.at[ab], gsem)]

    def scatters(tb, ib):
        del tb
        return [pltpu.make_async_copy(vals_v, acc_sh.at[dst_v.at[ib]], ssem)]

    pkv = [jnp.full((16,), 0.0, F32) + pk[k] for k in range(16)]
    qkv = [jnp.full((16,), 0.0, F32) + qk[k] for k in range(16)]
    rkv = [jnp.full((16,), 0.0, F32) + rk[k] for k in range(16)]

    def compute(buf):
        def group_body(g, carry2):
            gbase = g * 32
            for u in range(2):
                rows = iota + (gbase + u * 16)
                av = a_v[buf, pl.ds(gbase + u * 16, 16)]
                bv = b_v[buf, pl.ds(gbase + u * 16, 16)]
                for k in range(16):
                    col = jnp.maximum(av * pkv[k] + bv * qkv[k] + rkv[k], 0.0)
                    plsc.store_scatter(vals_v, [rows, kcols[k]], col)
            return carry2
        lax.fori_loop(0, CEC // 32, group_body, 0)

    for d in loads(0, 0):
        d.start()
    for d in loads(0, 0):
        d.wait()
    for d in gathers(0, 0):
        d.start()
    for d in loads(1, 1):
        d.start()
    for d in loads(1, 1):
        d.wait()
    for d in gathers(1, 1):
        d.start()
    for d in gathers(0, 0):
        d.wait()
    compute(0)
    for d in scatters(0, 0):
        d.start(add=True)
    for d in loads(2, 2):
        d.start()

    def chunk_body(t, carry):
        ib = lax.rem(t, 3)
        ib1 = lax.rem(t + 1, 3)
        ib2 = lax.rem(t + 2, 3)
        ab = lax.rem(t, 2)
        ab1 = lax.rem(t + 1, 2)
        tn1 = jnp.minimum(t + 1, NCHC - 1)
        tn2 = jnp.minimum(t + 2, NCHC - 1)
        for d in loads(tn1, ib1):
            d.wait()
        for d in gathers(tn1, ib1, ab1):
            d.start()
        for d in scatters(tn1, ib1):
            d.wait()
        for d in gathers(t, ib, ab):
            d.wait()
        compute(ab)
        for d in scatters(t, ib):
            d.start(add=True)
        for d in loads(tn2, ib2):
            d.start()
        return carry

    lax.fori_loop(1, NCHC, chunk_body, 0)
    for d in loads(0, lax.rem(NCHC + 1, 3)):
        d.wait()
    for d in gathers(0, lax.rem(NCHC, 3), lax.rem(NCHC, 2)):
        d.wait()
    for d in scatters(0, lax.rem(NCHC - 1, 3)):
        d.wait()
    plsc.subcore_barrier()
    pltpu.sync_copy(acc_sh.at[pl.ds(s * NT, NT), :],
                    agg_out.at[pl.ds(s * NT, NT), pl.ds(c * 16, 16)])


def _tc_z2_body(agg, deg3, a3, b3, pqr, w2l, w2r, b2l, z2_out, st_out, sacc):
    i = pl.program_id(0)
    deg = jnp.reshape(deg3[...], (RB, 1))
    a = jnp.reshape(a3[...], (RB, 1))
    b = jnp.reshape(b3[...], (RB, 1))
    p = pqr[0:1, :]
    q = pqr[1:2, :]
    r = pqr[2:3, :]
    h1 = jnp.maximum(a * p + b * q + r, 0.0)
    mean2 = agg[...] / jnp.maximum(deg, 1.0)
    z2 = (jnp.dot(mean2, w2l[...], preferred_element_type=F32)
          + jnp.dot(h1, w2r[...], preferred_element_type=F32) + b2l[...])
    valid = (lax.broadcasted_iota(I32, (RB, 1), 0) + i * RB) < N
    z2 = jnp.where(valid, z2, 0.0)
    z2_out[...] = z2

    @pl.when(i == 0)
    def _():
        sacc[...] = jnp.zeros((2, H), F32)

    sacc[0:1, :] += jnp.sum(z2, axis=0, keepdims=True)
    sacc[1:2, :] += jnp.sum(z2 * z2, axis=0, keepdims=True)

    @pl.when(i == NB - 1)
    def _():
        st_out[...] = sacc[...]


def _tc_z2(agg, deg3, a3, b3, pqr, w2l, w2r, b2l):
    return pl.pallas_call(
        _tc_z2_body,
        grid=(NB,),
        in_specs=[
            pl.BlockSpec((RB, H), lambda i: (i, 0)),
            pl.BlockSpec((1, RB, 1), lambda i: (i, 0, 0)),
            pl.BlockSpec((1, RB, 1), lambda i: (i, 0, 0)),
            pl.BlockSpec((1, RB, 1), lambda i: (i, 0, 0)),
            pl.BlockSpec((3, H), lambda i: (0, 0)),
            pl.BlockSpec((H, H), lambda i: (0, 0)),
            pl.BlockSpec((H, H), lambda i: (0, 0)),
            pl.BlockSpec((1, H), lambda i: (0, 0)),
        ],
        out_specs=[
            pl.BlockSpec((RB, H), lambda i: (i, 0)),
            pl.BlockSpec((2, H), lambda i: (0, 0)),
        ],
        out_shape=[jax.ShapeDtypeStruct((NP, H), F32),
                   jax.ShapeDtypeStruct((2, H), F32)],
        scratch_shapes=[pltpu.VMEM((2, H), F32)],
    )(agg, deg3, a3, b3, pqr, w2l, w2r, b2l)


def _tc_pool_body(z2, st, bat3, g2, be2, wg, bg, wo, bo, out,
                  gmax_s, den_s, num_s):
    ph = pl.program_id(0)
    i = pl.program_id(1)
    n = jnp.float32(N)
    mean = st[0:1, :] / n
    var = st[1:2, :] / n - mean * mean
    s2 = g2[...] * lax.rsqrt(var + 1e-5)
    t2 = be2[...] - mean * s2
    h2 = jnp.maximum(z2[...] * s2 + t2, 0.0)
    gate = jnp.dot(h2, wg[...], preferred_element_type=F32) + bg[...]
    bat = jnp.reshape(bat3[...], (RB, 1))
    gid = lax.broadcasted_iota(I32, (RB, G), 1).astype(F32)
    onehot = bat == gid
    onef = onehot.astype(F32)

    @pl.when(ph == 0)
    def _():
        @pl.when(i == 0)
        def _():
            gmax_s[...] = jnp.full((1, G), -jnp.inf, F32)
        masked = jnp.where(onehot, gate, -jnp.inf)
        bm = jnp.max(masked, axis=0, keepdims=True)
        gmax_s[...] = jnp.maximum(gmax_s[...], bm)

    @pl.when(ph == 1)
    def _():
        @pl.when(i == 0)
        def _():
            g0 = gmax_s[...]
            gmax_s[...] = jnp.where(jnp.isfinite(g0), g0, 0.0)
            den_s[...] = jnp.zeros((G, 1), F32)
            num_s[...] = jnp.zeros((G, H), F32)
        rowg = jnp.sum(jnp.where(onehot, gmax_s[...], 0.0),
                       axis=1, keepdims=True)
        e = jnp.exp(gate - rowg)
        e = jnp.where(bat < jnp.float32(G), e, 0.0)
        dn = (((0,), (0,)), ((), ()))
        den_s[...] += lax.dot_general(onef, e, dn, preferred_element_type=F32)
        num_s[...] += lax.dot_general(onef, e * h2, dn,
                                      preferred_element_type=F32)

        @pl.when(i == NB - 1)
        def _():
            pooled = num_s[...] / jnp.maximum(den_s[...], 1e-16)
            logit = jnp.dot(pooled, wo[...], preferred_element_type=F32) + bo[...]
            out[...] = jax.nn.sigmoid(logit)


def _tc_pool(z2, st, bat3, g2, be2, wg, bg, wo, bo):
    return pl.pallas_call(
        _tc_pool_body,
        grid=(2, NB),
        in_specs=[
            pl.BlockSpec((RB, H), lambda p, i: (i, 0)),
            pl.BlockSpec((2, H), lambda p, i: (0, 0)),
            pl.BlockSpec((1, RB, 1), lambda p, i: (i, 0, 0)),
            pl.BlockSpec((1, H), lambda p, i: (0, 0)),
            pl.BlockSpec((1, H), lambda p, i: (0, 0)),
            pl.BlockSpec((H, 1), lambda p, i: (0, 0)),
            pl.BlockSpec((1, 1), lambda p, i: (0, 0)),
            pl.BlockSpec((H, 1), lambda p, i: (0, 0)),
            pl.BlockSpec((1, 1), lambda p, i: (0, 0)),
        ],
        out_specs=pl.BlockSpec((G, 1), lambda p, i: (0, 0)),
        out_shape=jax.ShapeDtypeStruct((G, 1), F32),
        scratch_shapes=[pltpu.VMEM((1, G), F32),
                        pltpu.VMEM((G, 1), F32),
                        pltpu.VMEM((G, H), F32)],
    )(z2, st, bat3, g2, be2, wg, bg, wo, bo)


def kernel(x, edge_index, batch, W1l, b1l, W1r, g1, be1,
           W2l, b2l, W2r, g2, be2, Wg, bg, Wo, bo):
    xf = jnp.reshape(x, (N,))
    xpad = jnp.pad(xf, (0, NP - N))
    edge2 = edge_index.reshape(2, NBLK, 128)
    zerosT = jnp.zeros((NPT,), F32)
    ones128 = jnp.ones((128,), F32)
    zacc = jnp.zeros((NT, 16), F32)

    s1p, degp = _sc_stats(edge2, xpad, zerosT, ones128)

    a_arr, deg_arr, pqr = _tc_prep(
        s1p.reshape(2, 800, 128), degp.reshape(2, 800, 128),
        xpad.reshape(800, 128), W1l, jnp.reshape(b1l, (1, H)), W1r,
        jnp.reshape(g1, (1, H)), jnp.reshape(be1, (1, H)))

    agg = _sc_agg(edge_index, a_arr.reshape(NP)[:N], xf, pqr, zacc)

    deg3 = deg_arr.reshape(NB, RB, 1)
    a3 = a_arr.reshape(NB, RB, 1)
    b3 = xpad.reshape(NB, RB, 1)
    z2, st = _tc_z2(agg, deg3, a3, b3, pqr, W2l, W2r,
                    jnp.reshape(b2l, (1, H)))

    bat3 = jnp.pad(batch, (0, NP - N), constant_values=G).astype(F32)
    bat3 = bat3.reshape(NB, RB, 1)
    out = _tc_pool(z2, st, bat3, jnp.reshape(g2, (1, H)),
                   jnp.reshape(be2, (1, H)), Wg, jnp.reshape(bg, (1, 1)),
                   Wo, jnp.reshape(bo, (1, 1)))
    return out[:, 0]


# speedup vs baseline: 1.4373x; 1.0115x over previous
import functools

import jax
import jax.numpy as jnp
from jax import lax
from jax.experimental import pallas as pl
from jax.experimental.pallas import tpu as pltpu
from jax.experimental.pallas import tpu_sc as plsc

N = 100000
E = 6400000
G = 512
H = 32
NP = 102400
NB = 50
RB = 2048
NCHUNK = 2000
NBLK = 50000
CE = 3200
SUB = 25
CEC = 640
SUBC = 5
NCHC = 625
NPT = NP // 16
F32 = jnp.float32
I32 = jnp.int32

_mesh = plsc.VectorSubcoreMesh(core_axis_name="c", subcore_axis_name="s",
                               num_cores=2, num_subcores=16)


@functools.partial(
    pl.kernel, mesh=_mesh,
    compiler_params=pltpu.CompilerParams(use_tc_tiling_on_sc=False, needs_layout_passes=False),
    out_type=(jax.ShapeDtypeStruct((2, NP), F32),
              jax.ShapeDtypeStruct((2, NP), F32)),
    scratch_types=[
        pltpu.VMEM_SHARED((NP,), F32),
        pltpu.VMEM_SHARED((NP,), F32),
        pltpu.VMEM_SHARED((NP,), F32),
        pltpu.VMEM((2, SUB, 128), I32),
        pltpu.VMEM((2, SUB, 128), I32),
        pltpu.VMEM((2, SUB, 128), F32),
        pltpu.VMEM((128,), F32),
        pltpu.SemaphoreType.DMA,
        pltpu.SemaphoreType.DMA,
        pltpu.SemaphoreType.DMA,
        pltpu.SemaphoreType.DMA,
    ])
def _sc_stats(edge2, xpad, zerosT, ones128, s1_out, deg_out,
              x_sh, s1_sh, deg_sh, src_v, dst_v, xs_v, ones_v,
              lsem, gsem, asem, bsem):
    c = lax.axis_index("c")
    s = lax.axis_index("s")

    @pl.when(s == 0)
    def _():
        pltpu.sync_copy(xpad, x_sh)

    pltpu.sync_copy(zerosT, s1_sh.at[pl.ds(s * NPT, NPT)])
    pltpu.sync_copy(zerosT, deg_sh.at[pl.ds(s * NPT, NPT)])
    pltpu.sync_copy(ones128, ones_v)
    plsc.subcore_barrier()

    w = s * 2 + c
    start = w * 62 + jnp.minimum(w, 16)
    count = 62 + jnp.where(w < 16, 1, 0)

    def chunk_body(t, carry):
        cb = (start + t) * SUB
        pltpu.sync_copy(edge2.at[0, pl.ds(cb, SUB), :], src_v.at[0])
        pltpu.sync_copy(edge2.at[1, pl.ds(cb, SUB), :], dst_v.at[0])
        gds = [pltpu.async_copy(x_sh.at[src_v.at[0, j]], xs_v.at[0, j], gsem)
               for j in range(SUB)]
        for d in gds:
            d.wait()
        sds = []
        for j in range(SUB):
            sds.append(pltpu.async_copy(xs_v.at[0, j], s1_sh.at[dst_v.at[0, j]],
                                        asem, add=True))
            sds.append(pltpu.async_copy(ones_v, deg_sh.at[dst_v.at[0, j]],
                                        bsem, add=True))
        for d in sds:
            d.wait()
        return carry

    lax.fori_loop(0, count, chunk_body, 0)
    plsc.subcore_barrier()
    pltpu.sync_copy(s1_sh.at[pl.ds(s * NPT, NPT)],
                    s1_out.at[c, pl.ds(s * NPT, NPT)])
    pltpu.sync_copy(deg_sh.at[pl.ds(s * NPT, NPT)],
                    deg_out.at[c, pl.ds(s * NPT, NPT)])


def _tc_prep_body(s1p, degp, xp, w1l, b1l, w1r, g1, be1,
                  a_out, deg_out, pqr_out):
    s1 = s1p[0] + s1p[1]
    deg = degp[0] + degp[1]
    a = s1 / jnp.maximum(deg, 1.0)
    b = xp[...]
    n = jnp.float32(N)
    mua = jnp.sum(a) / n
    mub = jnp.sum(b) / n
    va = jnp.sum(a * a) / n - mua * mua
    vb = jnp.sum(b * b) / n - mub * mub
    cab = jnp.sum(a * b) / n - mua * mub
    u = w1l[...]
    v = w1r[...]
    c0 = b1l[...]
    mu = mua * u + mub * v + c0
    var = u * u * va + v * v * vb + 2.0 * u * v * cab
    inv = g1[...] * lax.rsqrt(var + 1e-5)
    p = u * inv
    q = v * inv
    r = (c0 - mu) * inv + be1[...]
    a_out[...] = a
    deg_out[...] = deg
    pqr_out[...] = jnp.concatenate([p, q, r], axis=0)


def _tc_prep(s1p, degp, xp, w1l, b1l, w1r, g1, be1):
    return pl.pallas_call(
        _tc_prep_body,
        out_shape=(jax.ShapeDtypeStruct((800, 128), F32),
                   jax.ShapeDtypeStruct((800, 128), F32),
                   jax.ShapeDtypeStruct((3, H), F32)),
    )(s1p, degp, xp, w1l, b1l, w1r, g1, be1)


NT = N // 16


@functools.partial(
    pl.kernel, mesh=_mesh,
    compiler_params=pltpu.CompilerParams(use_tc_tiling_on_sc=False, needs_layout_passes=False),
    out_type=jax.ShapeDtypeStruct((NP, H), F32),
    scratch_types=[
        pltpu.VMEM_SHARED((N,), F32),
        pltpu.VMEM_SHARED((N,), F32),
        pltpu.VMEM_SHARED((N, 16), F32),
        pltpu.VMEM((3, CEC), I32),
        pltpu.VMEM((3, CEC), I32),
        pltpu.VMEM((2, CEC), F32),
        pltpu.VMEM((2, CEC), F32),
        pltpu.VMEM((CEC, 16), F32),
        pltpu.VMEM((3, H), F32),
        pltpu.SemaphoreType.DMA,
        pltpu.SemaphoreType.DMA,
        pltpu.SemaphoreType.DMA,
    ])
def _sc_agg(edge_flat, a_hbm, b_hbm, pqr, zacc, agg_out,
            a_sh, b_sh, acc_sh, src_v, dst_v, a_v, b_v, vals_v, pqr_v,
            lsem, gsem, ssem):
    c = lax.axis_index("c")
    s = lax.axis_index("s")

    @pl.when(s == 0)
    def _():
        pltpu.sync_copy(a_hbm, a_sh)
        pltpu.sync_copy(b_hbm, b_sh)

    pltpu.sync_copy(pqr, pqr_v)
    pltpu.sync_copy(zacc, acc_sh.at[pl.ds(s * NT, NT), :])
    plsc.subcore_barrier()

    base_f = c * 16
    pv = pqr_v[0, pl.ds(base_f, 16)]
    qv = pqr_v[1, pl.ds(base_f, 16)]
    rv = pqr_v[2, pl.ds(base_f, 16)]
    pk = [pv[k] for k in range(16)]
    qk = [qv[k] for k in range(16)]
    rk = [rv[k] for k in range(16)]
    iota = lax.iota(I32, 16)
    kcols = [jnp.full((16,), k, I32) for k in range(16)]

    def loads(tb, ib):
        e0 = (s * NCHC + tb) * CEC
        return [pltpu.make_async_copy(edge_flat.at[0, pl.ds(e0, CEC)],
                                      src_v.at[ib], lsem),
                pltpu.make_async_copy(edge_flat.at[1, pl.ds(e0, CEC)],
                                      dst_v.at[ib], lsem)]

    def gathers(tb, ib, ab=0):
        del tb
        return [pltpu.make_async_copy(a_sh.at[src_v.at[ib]], a_v.at[ab], gsem),
                pltpu.make_async_copy(b_sh.at[src_v.at[ib]], b_v.at[ab], gsem)]

    def scatters(tb, ib):
        del tb
        return [pltpu.make_async_copy(vals_v, acc_sh.at[dst_v.at[ib]], ssem)]

    pkv = [jnp.full((16,), 0.0, F32) + pk[k] for k in range(16)]
    qkv = [jnp.full((16,), 0.0, F32) + qk[k] for k in range(16)]
    rkv = [jnp.full((16,), 0.0, F32) + rk[k] for k in range(16)]

    def compute(buf):
        def group_body(g, carry2):
            gbase = g * 64
            for u in range(4):
                rows = iota + (gbase + u * 16)
                av = a_v[buf, pl.ds(gbase + u * 16, 16)]
                bv = b_v[buf, pl.ds(gbase + u * 16, 16)]
                for k in range(16):
                    col = jnp.maximum(av * pkv[k] + bv * qkv[k] + rkv[k], 0.0)
                    plsc.store_scatter(vals_v, [rows, kcols[k]], col)
            return carry2
        lax.fori_loop(0, CEC // 64, group_body, 0)

    for d in loads(0, 0):
        d.start()
    for d in loads(0, 0):
        d.wait()
    for d in gathers(0, 0):
        d.start()
    for d in loads(1, 1):
        d.start()
    for d in loads(1, 1):
        d.wait()
    for d in gathers(1, 1):
        d.start()
    for d in gathers(0, 0):
        d.wait()
    compute(0)
    for d in scatters(0, 0):
        d.start(add=True)
    for d in loads(2, 2):
        d.start()

    def chunk_body(t, carry):
        ib = lax.rem(t, 3)
        ib1 = lax.rem(t + 1, 3)
        ib2 = lax.rem(t + 2, 3)
        ab = lax.rem(t, 2)
        ab1 = lax.rem(t + 1, 2)
        tn1 = jnp.minimum(t + 1, NCHC - 1)
        tn2 = jnp.minimum(t + 2, NCHC - 1)
        for d in loads(tn1, ib1):
            d.wait()
        for d in gathers(tn1, ib1, ab1):
            d.start()
        for d in scatters(tn1, ib1):
            d.wait()
        for d in gathers(t, ib, ab):
            d.wait()
        compute(ab)
        for d in scatters(t, ib):
            d.start(add=True)
        for d in loads(tn2, ib2):
            d.start()
        return carry

    lax.fori_loop(1, NCHC, chunk_body, 0)
    for d in loads(0, lax.rem(NCHC + 1, 3)):
        d.wait()
    for d in gathers(0, lax.rem(NCHC, 3), lax.rem(NCHC, 2)):
        d.wait()
    for d in scatters(0, lax.rem(NCHC - 1, 3)):
        d.wait()
    plsc.subcore_barrier()
    pltpu.sync_copy(acc_sh.at[pl.ds(s * NT, NT), :],
                    agg_out.at[pl.ds(s * NT, NT), pl.ds(c * 16, 16)])


def _tc_z2_body(agg, deg3, a3, b3, pqr, w2l, w2r, b2l, z2_out, st_out, sacc):
    i = pl.program_id(0)
    deg = jnp.reshape(deg3[...], (RB, 1))
    a = jnp.reshape(a3[...], (RB, 1))
    b = jnp.reshape(b3[...], (RB, 1))
    p = pqr[0:1, :]
    q = pqr[1:2, :]
    r = pqr[2:3, :]
    h1 = jnp.maximum(a * p + b * q + r, 0.0)
    mean2 = agg[...] / jnp.maximum(deg, 1.0)
    z2 = (jnp.dot(mean2, w2l[...], preferred_element_type=F32)
          + jnp.dot(h1, w2r[...], preferred_element_type=F32) + b2l[...])
    valid = (lax.broadcasted_iota(I32, (RB, 1), 0) + i * RB) < N
    z2 = jnp.where(valid, z2, 0.0)
    z2_out[...] = z2

    @pl.when(i == 0)
    def _():
        sacc[...] = jnp.zeros((2, H), F32)

    sacc[0:1, :] += jnp.sum(z2, axis=0, keepdims=True)
    sacc[1:2, :] += jnp.sum(z2 * z2, axis=0, keepdims=True)

    @pl.when(i == NB - 1)
    def _():
        st_out[...] = sacc[...]


def _tc_z2(agg, deg3, a3, b3, pqr, w2l, w2r, b2l):
    return pl.pallas_call(
        _tc_z2_body,
        grid=(NB,),
        in_specs=[
            pl.BlockSpec((RB, H), lambda i: (i, 0)),
            pl.BlockSpec((1, RB, 1), lambda i: (i, 0, 0)),
            pl.BlockSpec((1, RB, 1), lambda i: (i, 0, 0)),
            pl.BlockSpec((1, RB, 1), lambda i: (i, 0, 0)),
            pl.BlockSpec((3, H), lambda i: (0, 0)),
            pl.BlockSpec((H, H), lambda i: (0, 0)),
            pl.BlockSpec((H, H), lambda i: (0, 0)),
            pl.BlockSpec((1, H), lambda i: (0, 0)),
        ],
        out_specs=[
            pl.BlockSpec((RB, H), lambda i: (i, 0)),
            pl.BlockSpec((2, H), lambda i: (0, 0)),
        ],
        out_shape=[jax.ShapeDtypeStruct((NP, H), F32),
                   jax.ShapeDtypeStruct((2, H), F32)],
        scratch_shapes=[pltpu.VMEM((2, H), F32)],
    )(agg, deg3, a3, b3, pqr, w2l, w2r, b2l)


def _tc_pool_body(z2, st, bat3, g2, be2, wg, bg, wo, bo, out,
                  gmax_s, den_s, num_s):
    ph = pl.program_id(0)
    i = pl.program_id(1)
    n = jnp.float32(N)
    mean = st[0:1, :] / n
    var = st[1:2, :] / n - mean * mean
    s2 = g2[...] * lax.rsqrt(var + 1e-5)
    t2 = be2[...] - mean * s2
    h2 = jnp.maximum(z2[...] * s2 + t2, 0.0)
    gate = jnp.dot(h2, wg[...], preferred_element_type=F32) + bg[...]
    bat = jnp.reshape(bat3[...], (RB, 1))
    gid = lax.broadcasted_iota(I32, (RB, G), 1).astype(F32)
    onehot = bat == gid
    onef = onehot.astype(F32)

    @pl.when(ph == 0)
    def _():
        @pl.when(i == 0)
        def _():
            gmax_s[...] = jnp.full((1, G), -jnp.inf, F32)
        masked = jnp.where(onehot, gate, -jnp.inf)
        bm = jnp.max(masked, axis=0, keepdims=True)
        gmax_s[...] = jnp.maximum(gmax_s[...], bm)

    @pl.when(ph == 1)
    def _():
        @pl.when(i == 0)
        def _():
            g0 = gmax_s[...]
            gmax_s[...] = jnp.where(jnp.isfinite(g0), g0, 0.0)
            den_s[...] = jnp.zeros((G, 1), F32)
            num_s[...] = jnp.zeros((G, H), F32)
        rowg = jnp.sum(jnp.where(onehot, gmax_s[...], 0.0),
                       axis=1, keepdims=True)
        e = jnp.exp(gate - rowg)
        e = jnp.where(bat < jnp.float32(G), e, 0.0)
        dn = (((0,), (0,)), ((), ()))
        den_s[...] += lax.dot_general(onef, e, dn, preferred_element_type=F32)
        num_s[...] += lax.dot_general(onef, e * h2, dn,
                                      preferred_element_type=F32)

        @pl.when(i == NB - 1)
        def _():
            pooled = num_s[...] / jnp.maximum(den_s[...], 1e-16)
            logit = jnp.dot(pooled, wo[...], preferred_element_type=F32) + bo[...]
            out[...] = jax.nn.sigmoid(logit)


def _tc_pool(z2, st, bat3, g2, be2, wg, bg, wo, bo):
    return pl.pallas_call(
        _tc_pool_body,
        grid=(2, NB),
        in_specs=[
            pl.BlockSpec((RB, H), lambda p, i: (i, 0)),
            pl.BlockSpec((2, H), lambda p, i: (0, 0)),
            pl.BlockSpec((1, RB, 1), lambda p, i: (i, 0, 0)),
            pl.BlockSpec((1, H), lambda p, i: (0, 0)),
            pl.BlockSpec((1, H), lambda p, i: (0, 0)),
            pl.BlockSpec((H, 1), lambda p, i: (0, 0)),
            pl.BlockSpec((1, 1), lambda p, i: (0, 0)),
            pl.BlockSpec((H, 1), lambda p, i: (0, 0)),
            pl.BlockSpec((1, 1), lambda p, i: (0, 0)),
        ],
        out_specs=pl.BlockSpec((G, 1), lambda p, i: (0, 0)),
        out_shape=jax.ShapeDtypeStruct((G, 1), F32),
        scratch_shapes=[pltpu.VMEM((1, G), F32),
                        pltpu.VMEM((G, 1), F32),
                        pltpu.VMEM((G, H), F32)],
    )(z2, st, bat3, g2, be2, wg, bg, wo, bo)


def kernel(x, edge_index, batch, W1l, b1l, W1r, g1, be1,
           W2l, b2l, W2r, g2, be2, Wg, bg, Wo, bo):
    xf = jnp.reshape(x, (N,))
    xpad = jnp.pad(xf, (0, NP - N))
    edge2 = edge_index.reshape(2, NBLK, 128)
    zerosT = jnp.zeros((NPT,), F32)
    ones128 = jnp.ones((128,), F32)
    zacc = jnp.zeros((NT, 16), F32)

    s1p, degp = _sc_stats(edge2, xpad, zerosT, ones128)

    a_arr, deg_arr, pqr = _tc_prep(
        s1p.reshape(2, 800, 128), degp.reshape(2, 800, 128),
        xpad.reshape(800, 128), W1l, jnp.reshape(b1l, (1, H)), W1r,
        jnp.reshape(g1, (1, H)), jnp.reshape(be1, (1, H)))

    agg = _sc_agg(edge_index, a_arr.reshape(NP)[:N], xf, pqr, zacc)

    deg3 = deg_arr.reshape(NB, RB, 1)
    a3 = a_arr.reshape(NB, RB, 1)
    b3 = xpad.reshape(NB, RB, 1)
    z2, st = _tc_z2(agg, deg3, a3, b3, pqr, W2l, W2r,
                    jnp.reshape(b2l, (1, H)))

    bat3 = jnp.pad(batch, (0, NP - N), constant_values=G).astype(F32)
    bat3 = bat3.reshape(NB, RB, 1)
    out = _tc_pool(z2, st, bat3, jnp.reshape(g2, (1, H)),
                   jnp.reshape(be2, (1, H)), Wg, jnp.reshape(bg, (1, 1)),
                   Wo, jnp.reshape(bo, (1, 1)))
    return out[:, 0]
